# Initial kernel scaffold; baseline (speedup 1.0000x reference)
import jax
import jax.numpy as jnp
from jax.experimental import pallas as pl


def kernel(Eu, Ev, edge_index):
    raise NotImplementedError("write your pallas kernel here")


# speedup vs baseline: 3.7518x; 3.7518x over previous
import functools

import jax
import jax.numpy as jnp
from jax import lax
from jax.experimental import pallas as pl
from jax.experimental.pallas import tpu as pltpu
from jax.experimental.pallas import tpu_sc as plsc

_NC = 2
_NS = 16
_NW = _NC * _NS
_C = 512
_R = _C // 128


def _make_sc_kernel(d, e_pad):
  ew = e_pad // _NW
  nchunk = ew // _C
  rows_per_w = ew // 128

  mesh = plsc.VectorSubcoreMesh(core_axis_name="c", subcore_axis_name="s")

  @functools.partial(
      pl.kernel, mesh=mesh,
      compiler_params=pltpu.CompilerParams(
          needs_layout_passes=False, use_tc_tiling_on_sc=False),
      out_type=jax.ShapeDtypeStruct((e_pad,), jnp.float32),
      scratch_types=[
          pltpu.VMEM((_R, 128), jnp.int32),
          pltpu.VMEM((_R, 128), jnp.int32),
          pltpu.VMEM((_C, d), jnp.float32),
          pltpu.VMEM((_C, d), jnp.float32),
          pltpu.VMEM((_C,), jnp.float32),
          pltpu.SemaphoreType.DMA,
      ],
  )
  def k(src_hbm, dst_hbm, eu_hbm, ev_hbm, out_hbm, iu, iv, ru, rv, ov, sem):
    wid = lax.axis_index("s") * _NC + lax.axis_index("c")
    base_row = wid * rows_per_w
    base_edge = wid * ew

    def chunk(ci, carry):
      r0 = base_row + ci * _R
      pltpu.sync_copy(src_hbm.at[pl.ds(r0, _R)], iu)
      pltpu.sync_copy(dst_hbm.at[pl.ds(r0, _R)], iv)
      cps = []
      for j in range(_R):
        cps.append(pltpu.async_copy(
            eu_hbm.at[iu.at[j]], ru.at[pl.ds(j * 128, 128)], sem))
        cps.append(pltpu.async_copy(
            ev_hbm.at[iv.at[j]], rv.at[pl.ds(j * 128, 128)], sem))
      for cp in cps:
        cp.wait()

      lanes = lax.iota(jnp.int32, 16)

      def group(g, c2):
        e_vec = g * 16 + lanes
        acc = jnp.zeros((16,), jnp.float32)
        for dd in range(d):
          di = jnp.full((16,), dd, jnp.int32)
          gu = plsc.load_gather(ru, [e_vec, di])
          gv = plsc.load_gather(rv, [e_vec, di])
          acc = acc + gu * gv
        y = 1.0 / (1.0 + jnp.exp(-acc))
        ov[pl.ds(g * 16, 16)] = y
        return c2

      lax.fori_loop(0, _C // 16, group, 0)
      pltpu.sync_copy(ov, out_hbm.at[pl.ds(base_edge + ci * _C, _C)])
      return carry

    lax.fori_loop(0, nchunk, chunk, 0)

  return k


def kernel(Eu, Ev, edge_index):
  d = Eu.shape[1]
  e = edge_index.shape[1]
  e_pad = -(-e // (_NW * _C)) * (_NW * _C)
  idx = edge_index.astype(jnp.int32)
  pad = e_pad - e
  src = jnp.pad(idx[0], (0, pad)).reshape(e_pad // 128, 128)
  dst = jnp.pad(idx[1], (0, pad)).reshape(e_pad // 128, 128)
  out = _make_sc_kernel(d, e_pad)(src, dst, Eu, Ev)
  return out[:e]


# speedup vs baseline: 4.4892x; 1.1966x over previous
import functools

import jax
import jax.numpy as jnp
from jax import lax
from jax.experimental import pallas as pl
from jax.experimental.pallas import tpu as pltpu
from jax.experimental.pallas import tpu_sc as plsc

_NC = 2
_NS = 16
_NW = _NC * _NS
_C = 512
_R = _C // 128


def _make_sc_kernel(d, e_pad):
  ew = e_pad // _NW
  nchunk = ew // _C
  rows_per_w = ew // 128

  mesh = plsc.VectorSubcoreMesh(core_axis_name="c", subcore_axis_name="s")

  @functools.partial(
      pl.kernel, mesh=mesh,
      compiler_params=pltpu.CompilerParams(
          needs_layout_passes=False, use_tc_tiling_on_sc=False),
      out_type=jax.ShapeDtypeStruct((e_pad,), jnp.float32),
      scratch_types=[
          pltpu.VMEM((_R, 128), jnp.int32),
          pltpu.VMEM((_R, 128), jnp.int32),
          pltpu.VMEM((_R, 128), jnp.int32),
          pltpu.VMEM((_R, 128), jnp.int32),
          pltpu.VMEM((_C, d), jnp.float32),
          pltpu.VMEM((_C, d), jnp.float32),
          pltpu.VMEM((_C, d), jnp.float32),
          pltpu.VMEM((_C, d), jnp.float32),
          pltpu.VMEM((_C,), jnp.float32),
          pltpu.VMEM((_C,), jnp.float32),
          pltpu.SemaphoreType.DMA,
          pltpu.SemaphoreType.DMA,
          pltpu.SemaphoreType.DMA,
          pltpu.SemaphoreType.DMA,
          pltpu.SemaphoreType.DMA,
          pltpu.SemaphoreType.DMA,
      ],
  )
  def k(src_hbm, dst_hbm, eu_hbm, ev_hbm, out_hbm,
        iu0, iu1, iv0, iv1, ru0, ru1, rv0, rv1, ov0, ov1,
        is0, is1, rs0, rs1, os0, os1):
    iu = [iu0, iu1]
    iv = [iv0, iv1]
    ru = [ru0, ru1]
    rv = [rv0, rv1]
    ov = [ov0, ov1]
    isem = [is0, is1]
    rsem = [rs0, rs1]
    osem = [os0, os1]

    wid = lax.axis_index("s") * _NC + lax.axis_index("c")
    base_row = wid * rows_per_w
    base_edge = wid * ew
    lanes = lax.iota(jnp.int32, 16)

    def fetch_idx(ci, b):
      r0 = base_row + ci * _R
      a = pltpu.make_async_copy(src_hbm.at[pl.ds(r0, _R)], iu[b], isem[b])
      a.start()
      c = pltpu.make_async_copy(dst_hbm.at[pl.ds(r0, _R)], iv[b], isem[b])
      c.start()

    def wait_idx(b):
      pltpu.make_async_copy(src_hbm.at[pl.ds(0, _R)], iu[b], isem[b]).wait()
      pltpu.make_async_copy(dst_hbm.at[pl.ds(0, _R)], iv[b], isem[b]).wait()

    def fetch_rows(b):
      for j in range(_R):
        pltpu.make_async_copy(
            eu_hbm.at[iu[b].at[j]], ru[b].at[pl.ds(j * 128, 128)],
            rsem[b]).start()
        pltpu.make_async_copy(
            ev_hbm.at[iv[b].at[j]], rv[b].at[pl.ds(j * 128, 128)],
            rsem[b]).start()

    def wait_rows(b):
      pltpu.make_async_copy(eu_hbm.at[pl.ds(0, _C)], ru[b], rsem[b]).wait()
      pltpu.make_async_copy(ev_hbm.at[pl.ds(0, _C)], rv[b], rsem[b]).wait()

    def compute_store(ci, b):
      rub, rvb, ovb = ru[b], rv[b], ov[b]

      def group(g, c2):
        e_vec = g * 16 + lanes
        acc = jnp.zeros((16,), jnp.float32)
        for dd in range(d):
          di = jnp.full((16,), dd, jnp.int32)
          gu = plsc.load_gather(rub, [e_vec, di])
          gv = plsc.load_gather(rvb, [e_vec, di])
          acc = acc + gu * gv
        y = 1.0 / (1.0 + jnp.exp(-acc))
        ovb[pl.ds(g * 16, 16)] = y
        return c2

      lax.fori_loop(0, _C // 16, group, 0)
      pltpu.make_async_copy(
          ovb, out_hbm.at[pl.ds(base_edge + ci * _C, _C)], osem[b]).start()

    def wait_out(b):
      pltpu.make_async_copy(
          ov[b], out_hbm.at[pl.ds(0, _C)], osem[b]).wait()

    fetch_idx(0, 0)
    wait_idx(0)
    fetch_rows(0)
    fetch_idx(1, 1)

    def pair(i, carry):
      ci0 = i * 2
      for b in (0, 1):
        ci = ci0 + b
        nb = 1 - b

        @pl.when(ci + 1 < nchunk)
        def _():
          wait_idx(nb)
          fetch_rows(nb)

        wait_rows(b)

        @pl.when(ci + 2 < nchunk)
        def _():
          fetch_idx(ci + 2, b)

        @pl.when(ci >= 2)
        def _():
          wait_out(b)

        compute_store(ci, b)
      return carry

    lax.fori_loop(0, nchunk // 2, pair, 0)
    wait_out(0)
    wait_out(1)

  return k


def kernel(Eu, Ev, edge_index):
  d = Eu.shape[1]
  e = edge_index.shape[1]
  e_pad = -(-e // (_NW * _C)) * (_NW * _C)
  idx = edge_index.astype(jnp.int32)
  pad = e_pad - e
  src = jnp.pad(idx[0], (0, pad)).reshape(e_pad // 128, 128)
  dst = jnp.pad(idx[1], (0, pad)).reshape(e_pad // 128, 128)
  out = _make_sc_kernel(d, e_pad)(src, dst, Eu, Ev)
  return out[:e]


# speedup vs baseline: 11.3586x; 2.5302x over previous
import functools

import jax
import jax.numpy as jnp
from jax import lax
from jax.experimental import pallas as pl
from jax.experimental.pallas import tpu as pltpu
from jax.experimental.pallas import tpu_sc as plsc

_NC = 2
_NS = 16
_NW = _NC * _NS
_C = 512
_R = _C // 128


def _make_sc_kernel(d, e_pad):
  ew = e_pad // _NW
  nchunk = ew // _C
  rows_per_w = ew // 128

  mesh = plsc.VectorSubcoreMesh(core_axis_name="c", subcore_axis_name="s")

  @functools.partial(
      pl.kernel, mesh=mesh,
      compiler_params=pltpu.CompilerParams(
          needs_layout_passes=False, use_tc_tiling_on_sc=False),
      out_type=jax.ShapeDtypeStruct((e_pad,), jnp.float32),
      scratch_types=[
          pltpu.VMEM((_R, 128), jnp.int32),
          pltpu.VMEM((_R, 128), jnp.int32),
          pltpu.VMEM((_R, 128), jnp.int32),
          pltpu.VMEM((_R, 128), jnp.int32),
          pltpu.VMEM((_C, d), jnp.float32),
          pltpu.VMEM((_C, d), jnp.float32),
          pltpu.VMEM((_C, d), jnp.float32),
          pltpu.VMEM((_C, d), jnp.float32),
          pltpu.VMEM((_C,), jnp.float32),
          pltpu.VMEM((_C,), jnp.float32),
          pltpu.VMEM((16, 17), jnp.float32),
          pltpu.SemaphoreType.DMA,
          pltpu.SemaphoreType.DMA,
          pltpu.SemaphoreType.DMA,
          pltpu.SemaphoreType.DMA,
          pltpu.SemaphoreType.DMA,
          pltpu.SemaphoreType.DMA,
      ],
  )
  def k(src_hbm, dst_hbm, eu_hbm, ev_hbm, out_hbm,
        iu0, iu1, iv0, iv1, ru0, ru1, rv0, rv1, ov0, ov1, s1,
        is0, is1, rs0, rs1, os0, os1):
    iu = [iu0, iu1]
    iv = [iv0, iv1]
    ru = [ru0, ru1]
    rv = [rv0, rv1]
    ov = [ov0, ov1]
    isem = [is0, is1]
    rsem = [rs0, rs1]
    osem = [os0, os1]

    wid = lax.axis_index("s") * _NC + lax.axis_index("c")
    base_row = wid * rows_per_w
    base_edge = wid * ew
    lanes = lax.iota(jnp.int32, 16)

    def fetch_idx(ci, b):
      r0 = base_row + ci * _R
      a = pltpu.make_async_copy(src_hbm.at[pl.ds(r0, _R)], iu[b], isem[b])
      a.start()
      c = pltpu.make_async_copy(dst_hbm.at[pl.ds(r0, _R)], iv[b], isem[b])
      c.start()

    def wait_idx(b):
      pltpu.make_async_copy(src_hbm.at[pl.ds(0, _R)], iu[b], isem[b]).wait()
      pltpu.make_async_copy(dst_hbm.at[pl.ds(0, _R)], iv[b], isem[b]).wait()

    def fetch_rows(b):
      for j in range(_R):
        pltpu.make_async_copy(
            eu_hbm.at[iu[b].at[j]], ru[b].at[pl.ds(j * 128, 128)],
            rsem[b]).start()
        pltpu.make_async_copy(
            ev_hbm.at[iv[b].at[j]], rv[b].at[pl.ds(j * 128, 128)],
            rsem[b]).start()

    def wait_rows(b):
      pltpu.make_async_copy(eu_hbm.at[pl.ds(0, _C)], ru[b], rsem[b]).wait()
      pltpu.make_async_copy(ev_hbm.at[pl.ds(0, _C)], rv[b], rsem[b]).wait()

    def compute_store(ci, b):
      rub, rvb, ovb = ru[b], rv[b], ov[b]
      half = d // 2

      def group(g, c2):
        e0 = g * 16
        for j in range(16):
          e = e0 + j
          p = (rub[e, pl.ds(0, half)] * rvb[e, pl.ds(0, half)]
               + rub[e, pl.ds(half, half)] * rvb[e, pl.ds(half, half)])
          s1[j, pl.ds(0, 16)] = p
        acc = plsc.load_gather(s1, [lanes, jnp.zeros((16,), jnp.int32)])
        for dd in range(1, 16):
          di = jnp.full((16,), dd, jnp.int32)
          acc = acc + plsc.load_gather(s1, [lanes, di])
        y = 1.0 / (1.0 + jnp.exp(-acc))
        ovb[pl.ds(e0, 16)] = y
        return c2

      lax.fori_loop(0, _C // 16, group, 0)
      pltpu.make_async_copy(
          ovb, out_hbm.at[pl.ds(base_edge + ci * _C, _C)], osem[b]).start()

    def wait_out(b):
      pltpu.make_async_copy(
          ov[b], out_hbm.at[pl.ds(0, _C)], osem[b]).wait()

    fetch_idx(0, 0)
    wait_idx(0)
    fetch_rows(0)
    fetch_idx(1, 1)

    def pair(i, carry):
      ci0 = i * 2
      for b in (0, 1):
        ci = ci0 + b
        nb = 1 - b

        @pl.when(ci + 1 < nchunk)
        def _():
          wait_idx(nb)
          fetch_rows(nb)

        wait_rows(b)

        @pl.when(ci + 2 < nchunk)
        def _():
          fetch_idx(ci + 2, b)

        @pl.when(ci >= 2)
        def _():
          wait_out(b)

        compute_store(ci, b)
      return carry

    lax.fori_loop(0, nchunk // 2, pair, 0)
    wait_out(0)
    wait_out(1)

  return k


def kernel(Eu, Ev, edge_index):
  d = Eu.shape[1]
  e = edge_index.shape[1]
  e_pad = -(-e // (_NW * _C)) * (_NW * _C)
  idx = edge_index.astype(jnp.int32)
  pad = e_pad - e
  src = jnp.pad(idx[0], (0, pad)).reshape(e_pad // 128, 128)
  dst = jnp.pad(idx[1], (0, pad)).reshape(e_pad // 128, 128)
  out = _make_sc_kernel(d, e_pad)(src, dst, Eu, Ev)
  return out[:e]


# speedup vs baseline: 15.2193x; 1.3399x over previous
import functools

import jax
import jax.numpy as jnp
from jax import lax
from jax.experimental import pallas as pl
from jax.experimental.pallas import tpu as pltpu
from jax.experimental.pallas import tpu_sc as plsc

_NC = 2
_NS = 16
_NW = _NC * _NS
_C = 512
_R = _C // 128


def _make_sc_kernel(d, e_pad):
  ew = e_pad // _NW
  nchunk = ew // _C
  rows_per_w = ew // 128

  mesh = plsc.VectorSubcoreMesh(core_axis_name="c", subcore_axis_name="s")

  @functools.partial(
      pl.kernel, mesh=mesh,
      compiler_params=pltpu.CompilerParams(
          needs_layout_passes=False, use_tc_tiling_on_sc=False),
      out_type=jax.ShapeDtypeStruct((e_pad,), jnp.float32),
      scratch_types=[
          pltpu.VMEM((_R, 128), jnp.int32),
          pltpu.VMEM((_R, 128), jnp.int32),
          pltpu.VMEM((_R, 128), jnp.int32),
          pltpu.VMEM((_R, 128), jnp.int32),
          pltpu.VMEM((_C, d), jnp.float32),
          pltpu.VMEM((_C, d), jnp.float32),
          pltpu.VMEM((_C, d), jnp.float32),
          pltpu.VMEM((_C, d), jnp.float32),
          pltpu.VMEM((_C,), jnp.float32),
          pltpu.VMEM((_C,), jnp.float32),
          pltpu.VMEM((16, 17), jnp.float32),
          pltpu.SemaphoreType.DMA,
          pltpu.SemaphoreType.DMA,
          pltpu.SemaphoreType.DMA,
          pltpu.SemaphoreType.DMA,
          pltpu.SemaphoreType.DMA,
          pltpu.SemaphoreType.DMA,
      ],
  )
  def k(src_hbm, dst_hbm, eu_hbm, ev_hbm, out_hbm,
        iu0, iu1, iv0, iv1, ru0, ru1, rv0, rv1, ov0, ov1, s1,
        is0, is1, rs0, rs1, os0, os1):
    iu = [iu0, iu1]
    iv = [iv0, iv1]
    ru = [ru0, ru1]
    rv = [rv0, rv1]
    ov = [ov0, ov1]
    isem = [is0, is1]
    rsem = [rs0, rs1]
    osem = [os0, os1]

    wid = lax.axis_index("s") * _NC + lax.axis_index("c")
    base_row = wid * rows_per_w
    base_edge = wid * ew
    lanes = lax.iota(jnp.int32, 16)

    def fetch_idx(ci, b):
      r0 = base_row + ci * _R
      a = pltpu.make_async_copy(src_hbm.at[pl.ds(r0, _R)], iu[b], isem[b])
      a.start()
      c = pltpu.make_async_copy(dst_hbm.at[pl.ds(r0, _R)], iv[b], isem[b])
      c.start()

    def wait_idx(b):
      pltpu.make_async_copy(src_hbm.at[pl.ds(0, _R)], iu[b], isem[b]).wait()
      pltpu.make_async_copy(dst_hbm.at[pl.ds(0, _R)], iv[b], isem[b]).wait()

    def fetch_rows(b):
      for j in range(_R):
        pltpu.make_async_copy(
            eu_hbm.at[iu[b].at[j]], ru[b].at[pl.ds(j * 128, 128)],
            rsem[b]).start()
        pltpu.make_async_copy(
            ev_hbm.at[iv[b].at[j]], rv[b].at[pl.ds(j * 128, 128)],
            rsem[b]).start()

    def wait_rows(b):
      pltpu.make_async_copy(eu_hbm.at[pl.ds(0, _C)], ru[b], rsem[b]).wait()
      pltpu.make_async_copy(ev_hbm.at[pl.ds(0, _C)], rv[b], rsem[b]).wait()

    def compute_store(ci, b):
      rub, rvb, ovb = ru[b], rv[b], ov[b]
      half = d // 2

      def group(g, c2):
        e0 = g * 16
        for blk in range(4):
          es = [e0 + blk * 4 + i for i in range(4)]
          us = ([rub[e, pl.ds(0, half)] for e in es]
                + [rub[e, pl.ds(half, half)] for e in es])
          vs = ([rvb[e, pl.ds(0, half)] for e in es]
                + [rvb[e, pl.ds(half, half)] for e in es])
          ps = [u * v for u, v in zip(us, vs)]
          for i in range(4):
            s1[blk * 4 + i, pl.ds(0, 16)] = ps[i] + ps[4 + i]
        gs = [plsc.load_gather(s1, [lanes, jnp.full((16,), dd, jnp.int32)])
              for dd in range(16)]
        while len(gs) > 1:
          gs = [gs[i] + gs[i + 1] for i in range(0, len(gs), 2)]
        y = 1.0 / (1.0 + jnp.exp(-gs[0]))
        ovb[pl.ds(e0, 16)] = y
        return c2

      lax.fori_loop(0, _C // 16, group, 0)
      pltpu.make_async_copy(
          ovb, out_hbm.at[pl.ds(base_edge + ci * _C, _C)], osem[b]).start()

    def wait_out(b):
      pltpu.make_async_copy(
          ov[b], out_hbm.at[pl.ds(0, _C)], osem[b]).wait()

    fetch_idx(0, 0)
    wait_idx(0)
    fetch_rows(0)
    fetch_idx(1, 1)

    def pair(i, carry):
      ci0 = i * 2
      for b in (0, 1):
        ci = ci0 + b
        nb = 1 - b

        @pl.when(ci + 1 < nchunk)
        def _():
          wait_idx(nb)
          fetch_rows(nb)

        wait_rows(b)

        @pl.when(ci + 2 < nchunk)
        def _():
          fetch_idx(ci + 2, b)

        @pl.when(ci >= 2)
        def _():
          wait_out(b)

        compute_store(ci, b)
      return carry

    lax.fori_loop(0, nchunk // 2, pair, 0)
    wait_out(0)
    wait_out(1)

  return k


def kernel(Eu, Ev, edge_index):
  d = Eu.shape[1]
  e = edge_index.shape[1]
  e_pad = -(-e // (_NW * _C)) * (_NW * _C)
  idx = edge_index.astype(jnp.int32)
  pad = e_pad - e
  src = jnp.pad(idx[0], (0, pad)).reshape(e_pad // 128, 128)
  dst = jnp.pad(idx[1], (0, pad)).reshape(e_pad // 128, 128)
  out = _make_sc_kernel(d, e_pad)(src, dst, Eu, Ev)
  return out[:e]
